# Initial kernel scaffold; baseline (speedup 1.0000x reference)
import functools

import jax
import jax.numpy as jnp
from jax import lax
from jax.experimental import pallas as pl
from jax.experimental.pallas import tpu as pltpu
from jax.experimental.pallas import tpu_sc as plsc

N = 10000
D = 128
E = 320000
NC = 2
NS = 16
NW = NC * NS
PN = 10240
EW = E // NW
CPW = 80
EWP = CPW * 128
RB = 10
BR = PN // RB

_mesh = plsc.VectorSubcoreMesh(core_axis_name="c", subcore_axis_name="s")


def _hop_body(cur, al, ar, srcw, dstw, num_out, den_out,
              src_v, dst_v, al_v, ar_v, den_v, rows_v, e_v, stage_v, num_sh, sem):
    cid = lax.axis_index("c")
    sid = lax.axis_index("s")
    wid = cid * NS + sid
    rows_per_sub = PN // NS

    pltpu.sync_copy(srcw.at[wid], src_v)
    pltpu.sync_copy(dstw.at[wid], dst_v)
    pltpu.sync_copy(al, al_v)
    pltpu.sync_copy(ar, ar_v)

    zeros16 = jnp.zeros((16,), jnp.float32)

    def _zero_den(i, carry):
        den_v[pl.ds(i * 16, 16)] = zeros16
        return carry

    lax.fori_loop(0, PN // 16, _zero_den, 0)

    def _zero_stage(i, carry):
        for q in range(D // 16):
            stage_v[i, pl.ds(q * 16, 16)] = zeros16
        return carry

    lax.fori_loop(0, 128, _zero_stage, 0)

    for t in range(rows_per_sub // 128):
        pltpu.sync_copy(stage_v, num_sh.at[pl.ds(sid * rows_per_sub + t * 128, 128)])
    plsc.subcore_barrier()

    lane = lax.iota(jnp.int32, 16)

    def _chunk(c, carry):
        pltpu.async_copy(cur.at[src_v.at[c]], rows_v, sem).wait()
        for g in range(8):
            s16 = src_v[c, pl.ds(g * 16, 16)]
            d16 = dst_v[c, pl.ds(g * 16, 16)]
            a = plsc.load_gather(al_v, [s16]) + plsc.load_gather(ar_v, [d16])
            a = jnp.where(a >= 0.0, a, 0.2 * a)
            e = jnp.exp(a)
            pos = c * 128 + (g * 16) + lane
            e = jnp.where(pos < EW, e, 0.0)
            e_v[pl.ds(g * 16, 16)] = e
            plsc.addupdate_scatter(den_v, [d16], e)

        def _row(r, rc):
            eb = plsc.load_gather(e_v, [jnp.zeros((16,), jnp.int32) + r])
            for q in range(D // 16):
                rows_v[r, pl.ds(q * 16, 16)] = rows_v[r, pl.ds(q * 16, 16)] * eb
            return rc

        lax.fori_loop(0, 128, _row, 0)
        pltpu.sync_copy(rows_v, num_sh.at[dst_v.at[c]], add=True)
        return carry

    lax.fori_loop(0, CPW, _chunk, 0)

    pltpu.sync_copy(den_v, den_out.at[wid])
    plsc.subcore_barrier()

    for t in range(rows_per_sub // 128):
        base = sid * rows_per_sub + t * 128
        pltpu.sync_copy(num_sh.at[pl.ds(base, 128)], stage_v)
        pltpu.sync_copy(stage_v, num_out.at[cid, pl.ds(base, 128)])


_hop = pl.kernel(
    _hop_body,
    out_type=(
        jax.ShapeDtypeStruct((NC, PN, D), jnp.float32),
        jax.ShapeDtypeStruct((NW, PN), jnp.float32),
    ),
    mesh=_mesh,
    scratch_types=[
        pltpu.VMEM((CPW, 128), jnp.int32),
        pltpu.VMEM((CPW, 128), jnp.int32),
        pltpu.VMEM((PN,), jnp.float32),
        pltpu.VMEM((PN,), jnp.float32),
        pltpu.VMEM((PN,), jnp.float32),
        pltpu.VMEM((128, D), jnp.float32),
        pltpu.VMEM((128,), jnp.float32),
        pltpu.VMEM((128, D), jnp.float32),
        pltpu.VMEM_SHARED((PN, D), jnp.float32),
        pltpu.SemaphoreType.DMA,
    ],
)


def _lin_body(x_ref, w_ref, attl_ref, attr_ref, xl_ref, al_ref, ar_ref):
    xl = jnp.dot(x_ref[...], w_ref[...], preferred_element_type=jnp.float32)
    xl_ref[...] = xl
    al_ref[...] = jnp.sum(xl * attl_ref[...][None, :], axis=1)
    ar_ref[...] = jnp.sum(xl * attr_ref[...][None, :], axis=1)


def _lin(xp, W, attl, attr):
    return pl.pallas_call(
        _lin_body,
        grid=(RB,),
        in_specs=[
            pl.BlockSpec((BR, D), lambda i: (i, 0)),
            pl.BlockSpec((D, D), lambda i: (0, 0)),
            pl.BlockSpec((D,), lambda i: (0,)),
            pl.BlockSpec((D,), lambda i: (0,)),
        ],
        out_specs=[
            pl.BlockSpec((BR, D), lambda i: (i, 0)),
            pl.BlockSpec((BR,), lambda i: (i,)),
            pl.BlockSpec((BR,), lambda i: (i,)),
        ],
        out_shape=[
            jax.ShapeDtypeStruct((PN, D), jnp.float32),
            jax.ShapeDtypeStruct((PN,), jnp.float32),
            jax.ShapeDtypeStruct((PN,), jnp.float32),
        ],
    )(xp, W, attl, attr)


def _norm_body(num_ref, den_ref, attl_ref, attr_ref, cur_ref, al_ref, ar_ref):
    s = num_ref[0] + num_ref[1]
    dsum = jnp.sum(den_ref[...], axis=0)
    cur = s / (dsum + 1e-16)[:, None]
    cur_ref[...] = cur
    al_ref[...] = jnp.sum(cur * attl_ref[...][None, :], axis=1)
    ar_ref[...] = jnp.sum(cur * attr_ref[...][None, :], axis=1)


def _norm(num, den, attl, attr):
    return pl.pallas_call(
        _norm_body,
        grid=(RB,),
        in_specs=[
            pl.BlockSpec((NC, BR, D), lambda i: (0, i, 0)),
            pl.BlockSpec((NW, BR), lambda i: (0, i)),
            pl.BlockSpec((D,), lambda i: (0,)),
            pl.BlockSpec((D,), lambda i: (0,)),
        ],
        out_specs=[
            pl.BlockSpec((BR, D), lambda i: (i, 0)),
            pl.BlockSpec((BR,), lambda i: (i,)),
            pl.BlockSpec((BR,), lambda i: (i,)),
        ],
        out_shape=[
            jax.ShapeDtypeStruct((PN, D), jnp.float32),
            jax.ShapeDtypeStruct((PN,), jnp.float32),
            jax.ShapeDtypeStruct((PN,), jnp.float32),
        ],
    )(num, den, attl, attr)


def _comb_body(xl_ref, c1_ref, num_ref, den_ref, th_ref, b_ref, o_ref, *, do_elu):
    c2 = (num_ref[0] + num_ref[1]) / (jnp.sum(den_ref[...], axis=0) + 1e-16)[:, None]
    th = th_ref[...]
    out = (xl_ref[...] * (1.0 + th[0])[None, :]
           + c1_ref[...] * th[1][None, :]
           + c2 * th[2][None, :]
           + b_ref[...][None, :])
    if do_elu:
        out = jnp.where(out > 0.0, out, jnp.exp(out) - 1.0)
    o_ref[...] = out


def _comb(xl, c1, num, den, thp, b, do_elu):
    return pl.pallas_call(
        functools.partial(_comb_body, do_elu=do_elu),
        grid=(RB,),
        in_specs=[
            pl.BlockSpec((BR, D), lambda i: (i, 0)),
            pl.BlockSpec((BR, D), lambda i: (i, 0)),
            pl.BlockSpec((NC, BR, D), lambda i: (0, i, 0)),
            pl.BlockSpec((NW, BR), lambda i: (0, i)),
            pl.BlockSpec((8, D), lambda i: (0, 0)),
            pl.BlockSpec((D,), lambda i: (0,)),
        ],
        out_specs=pl.BlockSpec((BR, D), lambda i: (i, 0)),
        out_shape=jax.ShapeDtypeStruct((PN, D), jnp.float32),
    )(xl, c1, num, den, thp, b)


def _layer(xp, srcw, dstw, W, attl, attr, bias, theta, do_elu):
    xl, al, ar = _lin(xp, W, attl, attr)
    num1, den1 = _hop(xl, al, ar, srcw, dstw)
    cur1, al1, ar1 = _norm(num1, den1, attl, attr)
    num2, den2 = _hop(cur1, al1, ar1, srcw, dstw)
    thp = jnp.pad(theta, ((0, 8 - theta.shape[0]), (0, 0)))
    return _comb(xl, cur1, num2, den2, thp, bias, do_elu)


def kernel(x, edge_index, W1, att_l1, att_r1, bias1, theta1,
           W2, att_l2, att_r2, bias2, theta2):
    xp = jnp.pad(x, ((0, PN - N), (0, 0)))
    src = edge_index[0].astype(jnp.int32)
    dst = edge_index[1].astype(jnp.int32)
    srcw = jnp.pad(src.reshape(NW, EW), ((0, 0), (0, EWP - EW))).reshape(NW, CPW, 128)
    dstw = jnp.pad(dst.reshape(NW, EW), ((0, 0), (0, EWP - EW))).reshape(NW, CPW, 128)
    h = _layer(xp, srcw, dstw, W1, att_l1.reshape(-1), att_r1.reshape(-1),
               bias1, theta1, True)
    out = _layer(h, srcw, dstw, W2, att_l2.reshape(-1), att_r2.reshape(-1),
                 bias2, theta2, False)
    return out[:N]


# speedup vs baseline: 10.6631x; 10.6631x over previous
import functools

import jax
import jax.numpy as jnp
from jax import lax
from jax.experimental import pallas as pl
from jax.experimental.pallas import tpu as pltpu
from jax.experimental.pallas import tpu_sc as plsc

N = 10000
D = 128
E = 320000
NC = 2
NS = 16
NW = NC * NS
PN = 10240
EW = E // NW
CPW = 80
EWP = CPW * 128
G = 4
GRPS = CPW // G
RB = 10
BR = PN // RB

_mesh = plsc.VectorSubcoreMesh(core_axis_name="c", subcore_axis_name="s",
                               num_cores=NC, num_subcores=NS)


def _hop_body(cur, al, ar, srcw, dstw, num_out, den_out,
              src_g, dst_g, al_v, ar_v, den_v, rows_v, e_v, num_sh, sem):
    cid = lax.axis_index("c")
    sid = lax.axis_index("s")
    wid = cid * NS + sid
    rows_per_sub = PN // NS

    pltpu.sync_copy(al, al_v)
    pltpu.sync_copy(ar, ar_v)

    zeros16 = jnp.zeros((16,), jnp.float32)

    def _zero_den(i, carry):
        den_v[pl.ds(i * 16, 16)] = zeros16
        return carry

    lax.fori_loop(0, PN // 16, _zero_den, 0)

    def _zero_rows(i, carry):
        for q in range(D // 16):
            rows_v[i, pl.ds(q * 16, 16)] = zeros16
        return carry

    lax.fori_loop(0, 128, _zero_rows, 0)

    for t in range(rows_per_sub // 128):
        pltpu.sync_copy(rows_v, num_sh.at[pl.ds(sid * rows_per_sub + t * 128, 128)])
    plsc.subcore_barrier()

    lane = lax.iota(jnp.int32, 16)

    def _grp(grp, carry):
        pltpu.sync_copy(srcw.at[wid, pl.ds(grp * G, G)], src_g)
        pltpu.sync_copy(dstw.at[wid, pl.ds(grp * G, G)], dst_g)
        for g in range(G):
            c = grp * G + g
            pltpu.async_copy(cur.at[src_g.at[g]], rows_v, sem).wait()
            for q8 in range(8):
                s16 = src_g[g, pl.ds(q8 * 16, 16)]
                d16 = dst_g[g, pl.ds(q8 * 16, 16)]
                a = plsc.load_gather(al_v, [s16]) + plsc.load_gather(ar_v, [d16])
                a = jnp.where(a >= 0.0, a, 0.2 * a)
                e = jnp.exp(a)
                pos = c * 128 + (q8 * 16) + lane
                e = jnp.where(pos < EW, e, 0.0)
                e_v[pl.ds(q8 * 16, 16)] = e
                plsc.addupdate_scatter(den_v, [d16], e)

            def _row(r, rc):
                eb = plsc.load_gather(e_v, [jnp.zeros((16,), jnp.int32) + r])
                for q in range(D // 16):
                    rows_v[r, pl.ds(q * 16, 16)] = rows_v[r, pl.ds(q * 16, 16)] * eb
                return rc

            lax.fori_loop(0, 128, _row, 0)
            pltpu.sync_copy(rows_v, num_sh.at[dst_g.at[g]], add=True)
        return carry

    lax.fori_loop(0, GRPS, _grp, 0)

    pltpu.sync_copy(den_v, den_out.at[wid])
    plsc.subcore_barrier()

    for t in range(rows_per_sub // 128):
        base = sid * rows_per_sub + t * 128
        pltpu.sync_copy(num_sh.at[pl.ds(base, 128)], rows_v)
        pltpu.sync_copy(rows_v, num_out.at[cid, pl.ds(base, 128)])


_hop = pl.kernel(
    _hop_body,
    out_type=(
        jax.ShapeDtypeStruct((NC, PN, D), jnp.float32),
        jax.ShapeDtypeStruct((NW, PN), jnp.float32),
    ),
    mesh=_mesh,
    scratch_types=[
        pltpu.VMEM((G, 128), jnp.int32),
        pltpu.VMEM((G, 128), jnp.int32),
        pltpu.VMEM((PN,), jnp.float32),
        pltpu.VMEM((PN,), jnp.float32),
        pltpu.VMEM((PN,), jnp.float32),
        pltpu.VMEM((128, D), jnp.float32),
        pltpu.VMEM((128,), jnp.float32),
        pltpu.VMEM_SHARED((PN, D), jnp.float32),
        pltpu.SemaphoreType.DMA,
    ],
    compiler_params=pltpu.CompilerParams(needs_layout_passes=False),
)


def _lin_body(x_ref, w_ref, attl_ref, attr_ref, xl_ref, al_ref, ar_ref):
    xl = jnp.dot(x_ref[...], w_ref[...], preferred_element_type=jnp.float32)
    xl_ref[...] = xl
    al_ref[...] = jnp.sum(xl * attl_ref[...][None, :], axis=1)
    ar_ref[...] = jnp.sum(xl * attr_ref[...][None, :], axis=1)


def _lin(xp, W, attl, attr):
    return pl.pallas_call(
        _lin_body,
        grid=(RB,),
        in_specs=[
            pl.BlockSpec((BR, D), lambda i: (i, 0)),
            pl.BlockSpec((D, D), lambda i: (0, 0)),
            pl.BlockSpec((D,), lambda i: (0,)),
            pl.BlockSpec((D,), lambda i: (0,)),
        ],
        out_specs=[
            pl.BlockSpec((BR, D), lambda i: (i, 0)),
            pl.BlockSpec((BR,), lambda i: (i,)),
            pl.BlockSpec((BR,), lambda i: (i,)),
        ],
        out_shape=[
            jax.ShapeDtypeStruct((PN, D), jnp.float32),
            jax.ShapeDtypeStruct((PN,), jnp.float32),
            jax.ShapeDtypeStruct((PN,), jnp.float32),
        ],
    )(xp, W, attl, attr)


def _norm_body(num_ref, den_ref, attl_ref, attr_ref, cur_ref, al_ref, ar_ref):
    s = num_ref[0] + num_ref[1]
    dsum = jnp.sum(den_ref[...], axis=0)
    cur = s / (dsum + 1e-16)[:, None]
    cur_ref[...] = cur
    al_ref[...] = jnp.sum(cur * attl_ref[...][None, :], axis=1)
    ar_ref[...] = jnp.sum(cur * attr_ref[...][None, :], axis=1)


def _norm(num, den, attl, attr):
    return pl.pallas_call(
        _norm_body,
        grid=(RB,),
        in_specs=[
            pl.BlockSpec((NC, BR, D), lambda i: (0, i, 0)),
            pl.BlockSpec((NW, BR), lambda i: (0, i)),
            pl.BlockSpec((D,), lambda i: (0,)),
            pl.BlockSpec((D,), lambda i: (0,)),
        ],
        out_specs=[
            pl.BlockSpec((BR, D), lambda i: (i, 0)),
            pl.BlockSpec((BR,), lambda i: (i,)),
            pl.BlockSpec((BR,), lambda i: (i,)),
        ],
        out_shape=[
            jax.ShapeDtypeStruct((PN, D), jnp.float32),
            jax.ShapeDtypeStruct((PN,), jnp.float32),
            jax.ShapeDtypeStruct((PN,), jnp.float32),
        ],
    )(num, den, attl, attr)


def _comb_body(xl_ref, c1_ref, num_ref, den_ref, th_ref, b_ref, o_ref, *, do_elu):
    c2 = (num_ref[0] + num_ref[1]) / (jnp.sum(den_ref[...], axis=0) + 1e-16)[:, None]
    th = th_ref[...]
    out = (xl_ref[...] * (1.0 + th[0])[None, :]
           + c1_ref[...] * th[1][None, :]
           + c2 * th[2][None, :]
           + b_ref[...][None, :])
    if do_elu:
        out = jnp.where(out > 0.0, out, jnp.exp(out) - 1.0)
    o_ref[...] = out


def _comb(xl, c1, num, den, thp, b, do_elu):
    return pl.pallas_call(
        functools.partial(_comb_body, do_elu=do_elu),
        grid=(RB,),
        in_specs=[
            pl.BlockSpec((BR, D), lambda i: (i, 0)),
            pl.BlockSpec((BR, D), lambda i: (i, 0)),
            pl.BlockSpec((NC, BR, D), lambda i: (0, i, 0)),
            pl.BlockSpec((NW, BR), lambda i: (0, i)),
            pl.BlockSpec((8, D), lambda i: (0, 0)),
            pl.BlockSpec((D,), lambda i: (0,)),
        ],
        out_specs=pl.BlockSpec((BR, D), lambda i: (i, 0)),
        out_shape=jax.ShapeDtypeStruct((PN, D), jnp.float32),
    )(xl, c1, num, den, thp, b)


def _layer(xp, srcw, dstw, W, attl, attr, bias, theta, do_elu):
    xl, al, ar = _lin(xp, W, attl, attr)
    num1, den1 = _hop(xl, al, ar, srcw, dstw)
    cur1, al1, ar1 = _norm(num1, den1, attl, attr)
    num2, den2 = _hop(cur1, al1, ar1, srcw, dstw)
    thp = jnp.pad(theta, ((0, 8 - theta.shape[0]), (0, 0)))
    return _comb(xl, cur1, num2, den2, thp, bias, do_elu)


def kernel(x, edge_index, W1, att_l1, att_r1, bias1, theta1,
           W2, att_l2, att_r2, bias2, theta2):
    xp = jnp.pad(x, ((0, PN - N), (0, 0)))
    src = edge_index[0].astype(jnp.int32)
    dst = edge_index[1].astype(jnp.int32)
    srcw = jnp.pad(src.reshape(NW, EW), ((0, 0), (0, EWP - EW))).reshape(NW, CPW, 128)
    dstw = jnp.pad(dst.reshape(NW, EW), ((0, 0), (0, EWP - EW))).reshape(NW, CPW, 128)
    h = _layer(xp, srcw, dstw, W1, att_l1.reshape(-1), att_r1.reshape(-1),
               bias1, theta1, True)
    out = _layer(h, srcw, dstw, W2, att_l2.reshape(-1), att_r2.reshape(-1),
                 bias2, theta2, False)
    return out[:N]


# speedup vs baseline: 11.6308x; 1.0908x over previous
import functools

import jax
import jax.numpy as jnp
from jax import lax
from jax.experimental import pallas as pl
from jax.experimental.pallas import tpu as pltpu
from jax.experimental.pallas import tpu_sc as plsc

N = 10000
D = 128
E = 320000
NC = 2
NS = 16
NW = NC * NS
PN = 10240
EW = E // NW
CPW = 80
EWP = CPW * 128
G = 4
GRPS = CPW // G
RB = 10
BR = PN // RB

_mesh = plsc.VectorSubcoreMesh(core_axis_name="c", subcore_axis_name="s",
                               num_cores=NC, num_subcores=NS)


def _hop_body(cur, al, ar, srcw, dstw, num_out, den_out,
              src_g, dst_g, al_v, ar_v, den_v, rows_v, e_v, num_sh, sem):
    cid = lax.axis_index("c")
    sid = lax.axis_index("s")
    wid = cid * NS + sid
    rows_per_sub = PN // NS

    pltpu.sync_copy(al, al_v)
    pltpu.sync_copy(ar, ar_v)

    zeros16 = jnp.zeros((16,), jnp.float32)

    def _zero_den(i, carry):
        den_v[pl.ds(i * 16, 16)] = zeros16
        return carry

    lax.fori_loop(0, PN // 16, _zero_den, 0)

    def _zero_rows(i, carry):
        for q in range(D // 16):
            rows_v[i, pl.ds(q * 16, 16)] = zeros16
        return carry

    lax.fori_loop(0, 128, _zero_rows, 0)

    for t in range(rows_per_sub // 128):
        pltpu.sync_copy(rows_v, num_sh.at[pl.ds(sid * rows_per_sub + t * 128, 128)])
    plsc.subcore_barrier()

    lane = lax.iota(jnp.int32, 16)

    def _grp(grp, carry):
        pltpu.sync_copy(srcw.at[wid, pl.ds(grp * G, G)], src_g)
        pltpu.sync_copy(dstw.at[wid, pl.ds(grp * G, G)], dst_g)
        for g in range(G):
            c = grp * G + g
            pltpu.async_copy(cur.at[src_g.at[g]], rows_v, sem).wait()
            for q8 in range(8):
                s16 = src_g[g, pl.ds(q8 * 16, 16)]
                d16 = dst_g[g, pl.ds(q8 * 16, 16)]
                a = plsc.load_gather(al_v, [s16]) + plsc.load_gather(ar_v, [d16])
                a = jnp.where(a >= 0.0, a, 0.2 * a)
                e = jnp.exp(a)
                pos = c * 128 + (q8 * 16) + lane
                e = jnp.where(pos < EW, e, 0.0)
                e_v[pl.ds(q8 * 16, 16)] = e
                plsc.addupdate_scatter(den_v, [d16], e)

            def _row(i, rc):
                ev16 = e_v[pl.ds(i * 16, 16)]
                for u in range(16):
                    r = i * 16 + u
                    eb = jnp.broadcast_to(ev16[u], (16,))
                    for q in range(D // 16):
                        rows_v[r, pl.ds(q * 16, 16)] = rows_v[r, pl.ds(q * 16, 16)] * eb
                return rc

            lax.fori_loop(0, 8, _row, 0)
            pltpu.sync_copy(rows_v, num_sh.at[dst_g.at[g]], add=True)
        return carry

    lax.fori_loop(0, GRPS, _grp, 0)

    pltpu.sync_copy(den_v, den_out.at[wid])
    plsc.subcore_barrier()

    for t in range(rows_per_sub // 128):
        base = sid * rows_per_sub + t * 128
        pltpu.sync_copy(num_sh.at[pl.ds(base, 128)], rows_v)
        pltpu.sync_copy(rows_v, num_out.at[cid, pl.ds(base, 128)])


_hop = pl.kernel(
    _hop_body,
    out_type=(
        jax.ShapeDtypeStruct((NC, PN, D), jnp.float32),
        jax.ShapeDtypeStruct((NW, PN), jnp.float32),
    ),
    mesh=_mesh,
    scratch_types=[
        pltpu.VMEM((G, 128), jnp.int32),
        pltpu.VMEM((G, 128), jnp.int32),
        pltpu.VMEM((PN,), jnp.float32),
        pltpu.VMEM((PN,), jnp.float32),
        pltpu.VMEM((PN,), jnp.float32),
        pltpu.VMEM((128, D), jnp.float32),
        pltpu.VMEM((128,), jnp.float32),
        pltpu.VMEM_SHARED((PN, D), jnp.float32),
        pltpu.SemaphoreType.DMA,
    ],
    compiler_params=pltpu.CompilerParams(needs_layout_passes=False),
)


def _lin_body(x_ref, w_ref, attl_ref, attr_ref, xl_ref, al_ref, ar_ref):
    xl = jnp.dot(x_ref[...], w_ref[...], preferred_element_type=jnp.float32)
    xl_ref[...] = xl
    al_ref[...] = jnp.sum(xl * attl_ref[...][None, :], axis=1)
    ar_ref[...] = jnp.sum(xl * attr_ref[...][None, :], axis=1)


def _lin(xp, W, attl, attr):
    return pl.pallas_call(
        _lin_body,
        grid=(RB,),
        in_specs=[
            pl.BlockSpec((BR, D), lambda i: (i, 0)),
            pl.BlockSpec((D, D), lambda i: (0, 0)),
            pl.BlockSpec((D,), lambda i: (0,)),
            pl.BlockSpec((D,), lambda i: (0,)),
        ],
        out_specs=[
            pl.BlockSpec((BR, D), lambda i: (i, 0)),
            pl.BlockSpec((BR,), lambda i: (i,)),
            pl.BlockSpec((BR,), lambda i: (i,)),
        ],
        out_shape=[
            jax.ShapeDtypeStruct((PN, D), jnp.float32),
            jax.ShapeDtypeStruct((PN,), jnp.float32),
            jax.ShapeDtypeStruct((PN,), jnp.float32),
        ],
    )(xp, W, attl, attr)


def _norm_body(num_ref, den_ref, attl_ref, attr_ref, cur_ref, al_ref, ar_ref):
    s = num_ref[0] + num_ref[1]
    dsum = jnp.sum(den_ref[...], axis=0)
    cur = s / (dsum + 1e-16)[:, None]
    cur_ref[...] = cur
    al_ref[...] = jnp.sum(cur * attl_ref[...][None, :], axis=1)
    ar_ref[...] = jnp.sum(cur * attr_ref[...][None, :], axis=1)


def _norm(num, den, attl, attr):
    return pl.pallas_call(
        _norm_body,
        grid=(RB,),
        in_specs=[
            pl.BlockSpec((NC, BR, D), lambda i: (0, i, 0)),
            pl.BlockSpec((NW, BR), lambda i: (0, i)),
            pl.BlockSpec((D,), lambda i: (0,)),
            pl.BlockSpec((D,), lambda i: (0,)),
        ],
        out_specs=[
            pl.BlockSpec((BR, D), lambda i: (i, 0)),
            pl.BlockSpec((BR,), lambda i: (i,)),
            pl.BlockSpec((BR,), lambda i: (i,)),
        ],
        out_shape=[
            jax.ShapeDtypeStruct((PN, D), jnp.float32),
            jax.ShapeDtypeStruct((PN,), jnp.float32),
            jax.ShapeDtypeStruct((PN,), jnp.float32),
        ],
    )(num, den, attl, attr)


def _comb_body(xl_ref, c1_ref, num_ref, den_ref, th_ref, b_ref, o_ref, *, do_elu):
    c2 = (num_ref[0] + num_ref[1]) / (jnp.sum(den_ref[...], axis=0) + 1e-16)[:, None]
    th = th_ref[...]
    out = (xl_ref[...] * (1.0 + th[0])[None, :]
           + c1_ref[...] * th[1][None, :]
           + c2 * th[2][None, :]
           + b_ref[...][None, :])
    if do_elu:
        out = jnp.where(out > 0.0, out, jnp.exp(out) - 1.0)
    o_ref[...] = out


def _comb(xl, c1, num, den, thp, b, do_elu):
    return pl.pallas_call(
        functools.partial(_comb_body, do_elu=do_elu),
        grid=(RB,),
        in_specs=[
            pl.BlockSpec((BR, D), lambda i: (i, 0)),
            pl.BlockSpec((BR, D), lambda i: (i, 0)),
            pl.BlockSpec((NC, BR, D), lambda i: (0, i, 0)),
            pl.BlockSpec((NW, BR), lambda i: (0, i)),
            pl.BlockSpec((8, D), lambda i: (0, 0)),
            pl.BlockSpec((D,), lambda i: (0,)),
        ],
        out_specs=pl.BlockSpec((BR, D), lambda i: (i, 0)),
        out_shape=jax.ShapeDtypeStruct((PN, D), jnp.float32),
    )(xl, c1, num, den, thp, b)


def _layer(xp, srcw, dstw, W, attl, attr, bias, theta, do_elu):
    xl, al, ar = _lin(xp, W, attl, attr)
    num1, den1 = _hop(xl, al, ar, srcw, dstw)
    cur1, al1, ar1 = _norm(num1, den1, attl, attr)
    num2, den2 = _hop(cur1, al1, ar1, srcw, dstw)
    thp = jnp.pad(theta, ((0, 8 - theta.shape[0]), (0, 0)))
    return _comb(xl, cur1, num2, den2, thp, bias, do_elu)


def kernel(x, edge_index, W1, att_l1, att_r1, bias1, theta1,
           W2, att_l2, att_r2, bias2, theta2):
    xp = jnp.pad(x, ((0, PN - N), (0, 0)))
    src = edge_index[0].astype(jnp.int32)
    dst = edge_index[1].astype(jnp.int32)
    srcw = jnp.pad(src.reshape(NW, EW), ((0, 0), (0, EWP - EW))).reshape(NW, CPW, 128)
    dstw = jnp.pad(dst.reshape(NW, EW), ((0, 0), (0, EWP - EW))).reshape(NW, CPW, 128)
    h = _layer(xp, srcw, dstw, W1, att_l1.reshape(-1), att_r1.reshape(-1),
               bias1, theta1, True)
    out = _layer(h, srcw, dstw, W2, att_l2.reshape(-1), att_r2.reshape(-1),
                 bias2, theta2, False)
    return out[:N]
